# baseline (speedup 1.0000x reference)
import jax
import jax.numpy as jnp
from jax import lax
from jax.experimental import pallas as pl
from jax.experimental.pallas import tpu as pltpu
from jax.experimental.pallas import tpu_sc as plsc

B = 16384
D = 128
HALF = 64
NC = 2
NS = 16
L = 16
NW = NC * NS
BPW = B // NW
GROUPS = BPW // L
MT_WORDS = 12 * HALF
TBL_WORDS = MT_WORDS + 24 * HALF


def _emb_body(tbl_hbm, months_hbm, hours_hbm, out_hbm,
              tbl_v, m_v, h_v, rows_v, sem):
    wid = lax.axis_index("s") * NC + lax.axis_index("c")
    base = wid * BPW
    copies = [
        pltpu.async_copy(tbl_hbm, tbl_v, sem),
        pltpu.async_copy(months_hbm.at[pl.ds(base, BPW)], m_v, sem),
        pltpu.async_copy(hours_hbm.at[pl.ds(base, BPW)], h_v, sem),
    ]
    for c in copies:
        c.wait()

    lane = lax.iota(jnp.int32, L)
    lanec = [lane + c for c in range(0, HALF, L)]
    spl = [jnp.full((L,), l, jnp.int32) for l in range(L)]

    PIPE = 12

    def build(g):
        mb = m_v[pl.ds(g * L, L)] * HALF
        hb = h_v[pl.ds(g * L, L)] * HALF + MT_WORDS
        gbase = g * (L * D)
        pend = []

        def drain():
            off, v = pend.pop(0)
            rows_v[pl.ds(off, L)] = v

        @pl.loop(0, L, step=2)
        def _(l):
            for li in range(2):
                sl = jnp.full((L,), 0, jnp.int32) + (l + li)
                bm = mb.at[sl].get(mode="promise_in_bounds")
                bh = hb.at[sl].get(mode="promise_in_bounds")
                dst = gbase + (l + li) * D
                for ci, c in enumerate(range(0, HALF, L)):
                    pend.append((dst + c,
                                 plsc.load_gather(tbl_v, [bm + lanec[ci]])))
                    pend.append((dst + HALF + c,
                                 plsc.load_gather(tbl_v, [bh + lanec[ci]])))
                    while len(pend) > PIPE:
                        drain()
            while pend:
                drain()

    NCHUNK = 8
    CG = GROUPS // NCHUNK
    CW = BPW * D // NCHUNK

    def _chunk_copy(c):
        return pltpu.make_async_copy(
            rows_v.at[pl.ds(c * CW, CW)],
            out_hbm.at[pl.ds(base * D + c * CW, CW)], sem)

    @pl.loop(0, GROUPS)
    def _(g):
        build(g)
        for c in range(NCHUNK - 1):
            @pl.when(g == (c + 1) * CG - 1)
            def _(c=c):
                _chunk_copy(c).start()

    pltpu.sync_copy(
        rows_v.at[pl.ds((NCHUNK - 1) * CW, CW)],
        out_hbm.at[pl.ds(base * D + (NCHUNK - 1) * CW, CW)])
    for c in range(NCHUNK - 1):
        _chunk_copy(c).wait()


def kernel(months, hours, month_table, hour_table):
    mesh = plsc.VectorSubcoreMesh(core_axis_name="c", subcore_axis_name="s")
    cp = pltpu.CompilerParams(needs_layout_passes=False, use_tc_tiling_on_sc=False,
                              disable_bounds_checks=True,
                              disable_semaphore_checks=True)
    run = pl.kernel(
        _emb_body,
        out_type=jax.ShapeDtypeStruct((B * D,), jnp.float32),
        mesh=mesh,
        scratch_types=[
            pltpu.VMEM((TBL_WORDS,), jnp.float32),
            pltpu.VMEM((BPW,), jnp.int32),
            pltpu.VMEM((BPW,), jnp.int32),
            pltpu.VMEM((BPW * D,), jnp.float32),
            pltpu.SemaphoreType.DMA,
        ],
        compiler_params=cp,
    )
    tbl = jnp.concatenate([month_table.reshape(-1), hour_table.reshape(-1)])
    out = run(tbl, months.astype(jnp.int32), hours.astype(jnp.int32))
    return out.reshape(B, D)


# speedup vs baseline: 1.0227x; 1.0227x over previous
import jax
import jax.numpy as jnp
from jax import lax
from jax.experimental import pallas as pl
from jax.experimental.pallas import tpu as pltpu
from jax.experimental.pallas import tpu_sc as plsc

B = 16384
D = 128
HALF = 64
NC = 2
NS = 16
L = 16
NW = NC * NS
BPW = B // NW
GROUPS = BPW // L
MT_WORDS = 12 * HALF
TBL_WORDS = MT_WORDS + 24 * HALF


def _emb_body(tbl_hbm, months_hbm, hours_hbm, out_hbm,
              tbl_v, m_v, h_v, rows_v, sem):
    wid = lax.axis_index("s") * NC + lax.axis_index("c")
    base = wid * BPW
    copies = [
        pltpu.async_copy(tbl_hbm, tbl_v, sem),
        pltpu.async_copy(months_hbm.at[pl.ds(base, BPW)], m_v, sem),
        pltpu.async_copy(hours_hbm.at[pl.ds(base, BPW)], h_v, sem),
    ]
    for c in copies:
        c.wait()

    lane = lax.iota(jnp.int32, L)
    lanec = [lane + c for c in range(0, HALF, L)]
    spl = [jnp.full((L,), l, jnp.int32) for l in range(L)]

    PIPE = 16

    def build(g):
        mb = m_v[pl.ds(g * L, L)] * HALF
        hb = h_v[pl.ds(g * L, L)] * HALF + MT_WORDS
        gbase = g * (L * D)
        pend = []

        def drain():
            off, v = pend.pop(0)
            rows_v[pl.ds(off, L)] = v

        for l in range(L):
            bm = mb.at[spl[l]].get(mode="promise_in_bounds")
            bh = hb.at[spl[l]].get(mode="promise_in_bounds")
            for ci, c in enumerate(range(0, HALF, L)):
                pend.append((gbase + l * D + c,
                             plsc.load_gather(tbl_v, [bm + lanec[ci]])))
                pend.append((gbase + l * D + HALF + c,
                             plsc.load_gather(tbl_v, [bh + lanec[ci]])))
                while len(pend) > PIPE:
                    drain()
        while pend:
            drain()

    NCHUNK = 16
    CG = GROUPS // NCHUNK
    CW = BPW * D // NCHUNK

    def _chunk_copy(c):
        return pltpu.make_async_copy(
            rows_v.at[pl.ds(c * CW, CW)],
            out_hbm.at[pl.ds(base * D + c * CW, CW)], sem)

    @pl.loop(0, GROUPS)
    def _(g):
        build(g)
        for c in range(NCHUNK - 1):
            @pl.when(g == (c + 1) * CG - 1)
            def _(c=c):
                _chunk_copy(c).start()

    pltpu.sync_copy(
        rows_v.at[pl.ds((NCHUNK - 1) * CW, CW)],
        out_hbm.at[pl.ds(base * D + (NCHUNK - 1) * CW, CW)])
    for c in range(NCHUNK - 1):
        _chunk_copy(c).wait()


def kernel(months, hours, month_table, hour_table):
    mesh = plsc.VectorSubcoreMesh(core_axis_name="c", subcore_axis_name="s")
    cp = pltpu.CompilerParams(needs_layout_passes=False, use_tc_tiling_on_sc=False,
                              disable_bounds_checks=True,
                              disable_semaphore_checks=True)
    run = pl.kernel(
        _emb_body,
        out_type=jax.ShapeDtypeStruct((B * D,), jnp.float32),
        mesh=mesh,
        scratch_types=[
            pltpu.VMEM((TBL_WORDS,), jnp.float32),
            pltpu.VMEM((BPW,), jnp.int32),
            pltpu.VMEM((BPW,), jnp.int32),
            pltpu.VMEM((BPW * D,), jnp.float32),
            pltpu.SemaphoreType.DMA,
        ],
        compiler_params=cp,
    )
    tbl = jnp.concatenate([month_table.reshape(-1), hour_table.reshape(-1)])
    out = run(tbl, months.astype(jnp.int32), hours.astype(jnp.int32))
    return out.reshape(B, D)


# speedup vs baseline: 1.0315x; 1.0086x over previous
import jax
import jax.numpy as jnp
from jax import lax
from jax.experimental import pallas as pl
from jax.experimental.pallas import tpu as pltpu
from jax.experimental.pallas import tpu_sc as plsc

B = 16384
D = 128
HALF = 64
NC = 2
NS = 16
L = 16
NW = NC * NS
BPW = B // NW
GROUPS = BPW // L
MT_WORDS = 12 * HALF
TBL_WORDS = MT_WORDS + 24 * HALF


def _emb_body(tbl_hbm, months_hbm, hours_hbm, out_hbm,
              tbl_v, m_v, h_v, rows_v, sem):
    wid = lax.axis_index("s") * NC + lax.axis_index("c")
    base = wid * BPW
    copies = [
        pltpu.async_copy(tbl_hbm, tbl_v, sem),
        pltpu.async_copy(months_hbm.at[pl.ds(base, BPW)], m_v, sem),
        pltpu.async_copy(hours_hbm.at[pl.ds(base, BPW)], h_v, sem),
    ]
    for c in copies:
        c.wait()

    lane = lax.iota(jnp.int32, L)
    lanec = [lane + c for c in range(0, HALF, L)]
    spl = [jnp.full((L,), l, jnp.int32) for l in range(L)]

    PIPE = 12

    def build(g):
        mb = m_v[pl.ds(g * L, L)] * HALF
        hb = h_v[pl.ds(g * L, L)] * HALF + MT_WORDS
        gbase = g * (L * D)
        pend = []

        def drain():
            off, v = pend.pop(0)
            rows_v[pl.ds(off, L)] = v

        for l in range(L):
            bm = mb.at[spl[l]].get(mode="promise_in_bounds")
            bh = hb.at[spl[l]].get(mode="promise_in_bounds")
            for ci, c in enumerate(range(0, HALF, L)):
                pend.append((gbase + l * D + c,
                             plsc.load_gather(tbl_v, [bm + lanec[ci]])))
                pend.append((gbase + l * D + HALF + c,
                             plsc.load_gather(tbl_v, [bh + lanec[ci]])))
                while len(pend) > PIPE:
                    drain()
        while pend:
            drain()

    NCHUNK = 8
    CG = GROUPS // NCHUNK
    CW = BPW * D // NCHUNK

    def _chunk_copy(c):
        return pltpu.make_async_copy(
            rows_v.at[pl.ds(c * CW, CW)],
            out_hbm.at[pl.ds(base * D + c * CW, CW)], sem)

    @pl.loop(0, GROUPS)
    def _(g):
        build(g)
        for c in range(NCHUNK - 1):
            @pl.when(g == (c + 1) * CG - 1)
            def _(c=c):
                _chunk_copy(c).start()

    pltpu.sync_copy(
        rows_v.at[pl.ds((NCHUNK - 1) * CW, CW)],
        out_hbm.at[pl.ds(base * D + (NCHUNK - 1) * CW, CW)])
    for c in range(NCHUNK - 1):
        _chunk_copy(c).wait()


def kernel(months, hours, month_table, hour_table):
    mesh = plsc.VectorSubcoreMesh(core_axis_name="c", subcore_axis_name="s")
    cp = pltpu.CompilerParams(needs_layout_passes=False, use_tc_tiling_on_sc=False,
                              disable_bounds_checks=True,
                              disable_semaphore_checks=True)
    run = pl.kernel(
        _emb_body,
        out_type=jax.ShapeDtypeStruct((B * D,), jnp.float32),
        mesh=mesh,
        scratch_types=[
            pltpu.VMEM((TBL_WORDS,), jnp.float32),
            pltpu.VMEM((BPW,), jnp.int32),
            pltpu.VMEM((BPW,), jnp.int32),
            pltpu.VMEM((BPW * D,), jnp.float32),
            pltpu.SemaphoreType.DMA,
        ],
        compiler_params=cp,
    )
    tbl = jnp.concatenate([month_table.reshape(-1), hour_table.reshape(-1)])
    out = run(tbl, months.astype(jnp.int32), hours.astype(jnp.int32))
    return out.reshape(B, D)
